# baseline (speedup 1.0000x reference)
import functools

import jax
import jax.numpy as jnp
from jax import lax
from jax.experimental import pallas as pl
from jax.experimental.pallas import tpu as pltpu
from jax.experimental.pallas import tpu_sc as plsc

N_ROWS = 1_000_000
D = 64
NC = 2
NS = 16
NW = NC * NS
CHUNK = 800
N_CHUNKS = N_ROWS // CHUNK
MAX_CHUNKS_PER_W = -(-N_CHUNKS // NW)


@functools.partial(
    pl.kernel,
    out_type=jax.ShapeDtypeStruct((N_ROWS, D), jnp.float32),
    mesh=plsc.VectorSubcoreMesh(core_axis_name="c", subcore_axis_name="s"),
    scratch_types=[
        pltpu.VMEM((CHUNK,), jnp.int32),
        pltpu.VMEM((CHUNK, D), jnp.float32),
        pltpu.SemaphoreType.DMA,
    ],
    compiler_params=pltpu.CompilerParams(use_tc_tiling_on_sc=False),
)
def _lookup(cat_hbm, table_hbm, out_hbm, idx_v, rows_v, sem):
    wid = lax.axis_index("s") * NC + lax.axis_index("c")

    def body(j, carry):
        g = wid + j * NW

        @pl.when(g < N_CHUNKS)
        def _():
            base = pl.multiple_of(g * CHUNK, CHUNK)
            pltpu.sync_copy(cat_hbm.at[pl.ds(base, CHUNK)], idx_v)
            pltpu.async_copy(table_hbm.at[idx_v], rows_v, sem).wait()
            pltpu.sync_copy(rows_v, out_hbm.at[pl.ds(base, CHUNK)])

        return carry

    lax.fori_loop(0, MAX_CHUNKS_PER_W, body, 0)


def kernel(catalogue, item_emb_weight):
    return _lookup(catalogue, item_emb_weight)


# speedup vs baseline: 1.0314x; 1.0314x over previous
import functools

import jax
import jax.numpy as jnp
from jax import lax
from jax.experimental import pallas as pl
from jax.experimental.pallas import tpu as pltpu
from jax.experimental.pallas import tpu_sc as plsc

N_ROWS = 1_000_000
D = 64
NC = 2
NS = 16
NW = NC * NS
CHUNK = 400
N_CHUNKS = N_ROWS // CHUNK
NBUF = 4
J_MAX = ((N_CHUNKS + NW - 1) // NW + NBUF - 1) // NBUF * NBUF


@functools.partial(
    pl.kernel,
    out_type=jax.ShapeDtypeStruct((N_ROWS, D), jnp.float32),
    mesh=plsc.VectorSubcoreMesh(core_axis_name="c", subcore_axis_name="s"),
    scratch_types=[
        [pltpu.VMEM((CHUNK,), jnp.int32) for _ in range(NBUF)],
        [pltpu.VMEM((CHUNK, D), jnp.float32) for _ in range(NBUF)],
        [pltpu.SemaphoreType.DMA for _ in range(NBUF)],
        [pltpu.SemaphoreType.DMA for _ in range(NBUF)],
    ],
    compiler_params=pltpu.CompilerParams(use_tc_tiling_on_sc=False),
)
def _lookup(cat_hbm, table_hbm, out_hbm, idx_v, rows_v, gsem, ssem):
    wid = lax.axis_index("s") * NC + lax.axis_index("c")

    def chunk_of(j):
        return wid + j * NW

    def valid(j):
        return chunk_of(j) < N_CHUNKS

    def base_of(j):
        return pl.multiple_of(chunk_of(j) * CHUNK, CHUNK)

    def start_gather(j, b):
        @pl.when(valid(j))
        def _():
            pltpu.sync_copy(cat_hbm.at[pl.ds(base_of(j), CHUNK)], idx_v[b])
            pltpu.async_copy(table_hbm.at[idx_v[b]], rows_v[b], gsem[b])

    for u in range(2):
        start_gather(u, u)

    def group(k, carry):
        for u in range(NBUF):
            j = NBUF * k + u
            b = u

            @pl.when(valid(j))
            def _(j=j, b=b):
                pltpu.make_async_copy(table_hbm.at[idx_v[b]], rows_v[b],
                                      gsem[b]).wait()
                pltpu.async_copy(rows_v[b], out_hbm.at[pl.ds(base_of(j), CHUNK)],
                                 ssem[b])

            @pl.when((j >= 2) & valid(j - 2))
            def _(j=j, b2=(u + 2) % NBUF):
                pltpu.make_async_copy(rows_v[b2],
                                      out_hbm.at[pl.ds(base_of(j - 2), CHUNK)],
                                      ssem[b2]).wait()

            start_gather(j + 2, (u + 2) % NBUF)
        return carry

    lax.fori_loop(0, J_MAX // NBUF, group, 0)

    for j in (J_MAX - 2, J_MAX - 1):
        @pl.when(valid(j))
        def _(j=j, b=j % NBUF):
            pltpu.make_async_copy(rows_v[b], out_hbm.at[pl.ds(base_of(j), CHUNK)],
                                  ssem[b]).wait()


def kernel(catalogue, item_emb_weight):
    return _lookup(catalogue, item_emb_weight)
